# Initial kernel scaffold; baseline (speedup 1.0000x reference)
import jax
import jax.numpy as jnp
from jax.experimental import pallas as pl


def kernel(x, edge_index, edge_type, weight, self_loop_w):
    raise NotImplementedError("write your pallas kernel here")


# speedup vs baseline: 28.2461x; 28.2461x over previous
import functools

import jax
import jax.numpy as jnp
from jax import lax
from jax.experimental import pallas as pl
from jax.experimental.pallas import tpu as pltpu
from jax.experimental.pallas import tpu_sc as plsc

_NC = 2
_NS = 16
_NW = _NC * _NS
_K = 128


def _matmul_body(x_ref, w_ref, y_ref):
    y_ref[...] = jnp.dot(x_ref[...], w_ref[0],
                         preferred_element_type=jnp.float32)[None]


def _type_matmuls(x, wcat):
    n, cin = x.shape
    t, _, cout = wcat.shape
    bn = 1000
    return pl.pallas_call(
        _matmul_body,
        grid=(t, n // bn),
        in_specs=[
            pl.BlockSpec((bn, cin), lambda ti, i: (i, 0)),
            pl.BlockSpec((1, cin, cout), lambda ti, i: (ti, 0, 0)),
        ],
        out_specs=pl.BlockSpec((1, bn, cout), lambda ti, i: (ti, i, 0)),
        out_shape=jax.ShapeDtypeStruct((t, n, cout), jnp.float32),
    )(x, wcat)


def _combine_body(p_ref, y_ref, o_ref):
    o_ref[...] = p_ref[0] + p_ref[1] + y_ref[0]


def _combine(partials, y, self_idx):
    _, n, cout = y.shape
    bn = 1000
    return pl.pallas_call(
        _combine_body,
        grid=(n // bn,),
        in_specs=[
            pl.BlockSpec((2, bn, cout), lambda i: (0, i, 0)),
            pl.BlockSpec((1, bn, cout), lambda i: (self_idx, i, 0)),
        ],
        out_specs=pl.BlockSpec((bn, cout), lambda i: (i, 0)),
        out_shape=jax.ShapeDtypeStruct((n, cout), jnp.float32),
    )(partials, y)


def _make_edge_kernel(n_nodes, cout, n_chunks):
    n_acc = n_nodes + _NS
    r0 = (n_nodes // _NS) // 8 * 8
    r_last = n_nodes - r0 * (_NS - 1)
    mesh = plsc.VectorSubcoreMesh(core_axis_name="c", subcore_axis_name="s")

    @functools.partial(
        pl.kernel,
        out_type=jax.ShapeDtypeStruct((_NC, n_nodes, cout), jnp.float32),
        mesh=mesh,
        scratch_types=[
            pltpu.VMEM((n_chunks, _K), jnp.int32),
            pltpu.VMEM((n_chunks, _K), jnp.int32),
            pltpu.VMEM((_K, cout), jnp.float32),
            pltpu.VMEM_SHARED((n_acc, cout), jnp.float32),
            pltpu.SemaphoreType.DMA,
        ],
    )
    def edge_kernel(y_hbm, gidx_hbm, dst_hbm, zeros_hbm, out_hbm,
                    gidx_v, dst_v, rows_v, acc, sem):
        cid = lax.axis_index("c")
        sid = lax.axis_index("s")
        wid = sid * _NC + cid

        @pl.when(sid < _NS - 1)
        def _():
            pltpu.sync_copy(zeros_hbm.at[pl.ds(sid * r0, r0)],
                            acc.at[pl.ds(sid * r0, r0)])

        @pl.when(sid == _NS - 1)
        def _():
            pltpu.sync_copy(zeros_hbm.at[pl.ds(r0 * (_NS - 1), r_last)],
                            acc.at[pl.ds(r0 * (_NS - 1), r_last)])
        pltpu.sync_copy(gidx_hbm.at[wid], gidx_v)
        pltpu.sync_copy(dst_hbm.at[wid], dst_v)
        plsc.subcore_barrier()

        def body(j, carry):
            pltpu.async_copy(y_hbm.at[gidx_v.at[j]], rows_v, sem).wait()
            pltpu.sync_copy(rows_v, acc.at[dst_v.at[j]], add=True)
            return carry

        lax.fori_loop(0, n_chunks, body, 0)
        plsc.subcore_barrier()

        @pl.when(sid < _NS - 1)
        def _():
            pltpu.sync_copy(acc.at[pl.ds(sid * r0, r0)],
                            out_hbm.at[cid, pl.ds(sid * r0, r0)])

        @pl.when(sid == _NS - 1)
        def _():
            pltpu.sync_copy(acc.at[pl.ds(r0 * (_NS - 1), r_last)],
                            out_hbm.at[cid, pl.ds(r0 * (_NS - 1), r_last)])

    return edge_kernel


def kernel(x, edge_index, edge_type, weight, self_loop_w):
    n, cin = x.shape
    n_types, _, cout = weight.shape
    e = edge_index.shape[1]
    x = x.astype(jnp.float32)
    src = edge_index[0].astype(jnp.int32)
    dst = edge_index[1].astype(jnp.int32)
    et = edge_type.astype(jnp.int32)

    wcat = jnp.concatenate(
        [weight.astype(jnp.float32), self_loop_w.T.astype(jnp.float32)[None]],
        axis=0)
    y = _type_matmuls(x, wcat)

    n_chunks = -(-e // (_NW * _K))
    e_pad = _NW * _K * n_chunks
    gidx = et * n + src
    gidx = jnp.concatenate(
        [gidx, jnp.zeros((e_pad - e,), jnp.int32)]).reshape(_NW, n_chunks, _K)
    dstp = jnp.concatenate(
        [dst, jnp.full((e_pad - e,), n, jnp.int32)]).reshape(_NW, n_chunks, _K)
    zeros = jnp.zeros((n, cout), jnp.float32)

    edge_kernel = _make_edge_kernel(n, cout, n_chunks)
    partials = edge_kernel(y.reshape((n_types + 1) * n, cout),
                           gidx, dstp, zeros)
    return _combine(partials, y, n_types)
